# baseline (speedup 1.0000x reference)
import functools

import jax
import jax.numpy as jnp
from jax import lax
from jax.experimental import pallas as pl
from jax.experimental.pallas import tpu as pltpu
from jax.experimental.pallas import tpu_sc as plsc

D_TABLE = 128
BLK = 128
NUM_CORES = 2
NUM_SUBCORES = 16
NW = NUM_CORES * NUM_SUBCORES
K = 2
NBUF = 3


@functools.partial(jax.jit, static_argnames=("n_blocks",))
def _sc_gather(idx2d, pe, n_blocks):
    blocks_per_w = n_blocks // NW
    G = blocks_per_w // K
    assert G >= 7 and (G - 4) % NBUF == 0
    mesh = plsc.VectorSubcoreMesh(core_axis_name="c", subcore_axis_name="s")

    @functools.partial(
        pl.kernel,
        mesh=mesh,
        out_type=jax.ShapeDtypeStruct((n_blocks, BLK, D_TABLE), jnp.float32),
        scratch_types=(
            [pltpu.VMEM((K, BLK), jnp.int32)] * NBUF
            + [pltpu.VMEM((K, BLK, D_TABLE), jnp.float32)] * NBUF
            + [pltpu.VMEM_SHARED((1024, D_TABLE), jnp.float32)]
            + [pltpu.SemaphoreType.DMA] * (3 * NBUF)
        ),
    )
    def k(idx_hbm, pe_hbm, out_hbm, ib0, ib1, ib2, rb0, rb1, rb2, pe_sh,
          gs0, gs1, gs2, ss0, ss1, ss2, is0, is1, is2):
        ibs = (ib0, ib1, ib2)
        rbs = (rb0, rb1, rb2)
        gsem = (gs0, gs1, gs2)
        ssem = (ss0, ss1, ss2)
        isem = (is0, is1, is2)
        wid = lax.axis_index("s") * NUM_CORES + lax.axis_index("c")
        base = wid * blocks_per_w

        @pl.when(lax.axis_index("s") == 0)
        def _stage():
            pltpu.sync_copy(pe_hbm, pe_sh)

        plsc.subcore_barrier()

        def fire_g(g, b):
            for j in range(K):
                pltpu.async_copy(pe_sh.at[ibs[b].at[j]], rbs[b].at[j], gsem[b])

        def fire_store(g, b):
            pltpu.async_copy(rbs[b], out_hbm.at[pl.ds(base + g * K, K)], ssem[b])

        def fire_idx(g, b):
            pltpu.async_copy(idx_hbm.at[pl.ds(base + g * K, K)], ibs[b], isem[b])

        def wait_store(b):
            pltpu.make_async_copy(rbs[b], out_hbm.at[pl.ds(base, K)], ssem[b]).wait()

        def wait_gathers(b):
            for j in range(K):
                pltpu.make_async_copy(
                    out_hbm.at[base].at[pl.ds(0, BLK)], rbs[b].at[j], gsem[b]
                ).wait()

        def wait_idx(b):
            pltpu.make_async_copy(idx_hbm.at[pl.ds(base, K)], ibs[b], isem[b]).wait()

        pltpu.sync_copy(idx_hbm.at[pl.ds(base, K)], ibs[0])
        fire_g(0, 0)
        fire_idx(1, 1)
        wait_idx(1)
        fire_g(1, 1)
        fire_idx(2, 2)
        wait_gathers(0)
        fire_store(0, 0)
        wait_idx(2)
        fire_g(2, 2)
        fire_idx(3, 0)
        wait_gathers(1)
        fire_store(1, 1)

        def body(t, carry):
            g0 = 2 + NBUF * t
            for db in range(NBUF):
                g = g0 + db
                b = (2 + db) % NBUF
                bn = (b + 1) % NBUF
                bp = (b + 2) % NBUF
                wait_store(bn)
                wait_idx(bn)
                fire_g(g + 1, bn)
                fire_idx(g + 2, bp)
                wait_gathers(b)
                fire_store(g, b)
            return carry

        lax.fori_loop(0, (G - 4) // NBUF, body, 0)

        bL = (G - 2) % NBUF
        wait_store((bL + 1) % NBUF)
        wait_idx((bL + 1) % NBUF)
        fire_g(G - 1, (bL + 1) % NBUF)
        wait_gathers(bL)
        fire_store(G - 2, bL)
        wait_gathers((bL + 1) % NBUF)
        fire_store(G - 1, (bL + 1) % NBUF)
        wait_store((bL + 2) % NBUF)
        wait_store(bL)
        wait_store((bL + 1) % NBUF)

    return k(idx2d, pe)


def kernel(index, pe):
    b, t = index.shape
    n = b * t
    n_blocks = n // BLK
    idx2d = index.reshape(n_blocks, BLK)
    out = _sc_gather(idx2d, pe, n_blocks)
    return out.reshape(b, t, D_TABLE)


# speedup vs baseline: 1.0007x; 1.0007x over previous
import functools

import jax
import jax.numpy as jnp
from jax import lax
from jax.experimental import pallas as pl
from jax.experimental.pallas import tpu as pltpu
from jax.experimental.pallas import tpu_sc as plsc

D_TABLE = 128
BLK = 128
NUM_CORES = 2
NUM_SUBCORES = 16
NW = NUM_CORES * NUM_SUBCORES
K = 2
NBUF = 3


@functools.partial(jax.jit, static_argnames=("n_blocks",))
def _sc_gather(idx2d, pe, n_blocks):
    blocks_per_w = n_blocks // NW
    G = blocks_per_w // K
    assert G >= 7 and (G - 4) % NBUF == 0
    mesh = plsc.VectorSubcoreMesh(core_axis_name="c", subcore_axis_name="s")

    @functools.partial(
        pl.kernel,
        mesh=mesh,
        out_type=jax.ShapeDtypeStruct((n_blocks, BLK, D_TABLE), jnp.float32),
        scratch_types=(
            [pltpu.VMEM((K, BLK), jnp.int32)] * NBUF
            + [pltpu.VMEM((K, BLK, D_TABLE), jnp.float32)] * NBUF
            + [pltpu.VMEM_SHARED((1024, D_TABLE), jnp.float32)]
            + [pltpu.SemaphoreType.DMA] * (3 * NBUF)
        ),
    )
    def k(idx_hbm, pe_hbm, out_hbm, ib0, ib1, ib2, rb0, rb1, rb2, pe_sh,
          gs0, gs1, gs2, ss0, ss1, ss2, is0, is1, is2):
        ibs = (ib0, ib1, ib2)
        rbs = (rb0, rb1, rb2)
        gsem = (gs0, gs1, gs2)
        ssem = (ss0, ss1, ss2)
        isem = (is0, is1, is2)
        wid = lax.axis_index("s") * NUM_CORES + lax.axis_index("c")
        base = wid * blocks_per_w

        @pl.when(lax.axis_index("s") == 0)
        def _stage():
            pltpu.sync_copy(pe_hbm, pe_sh)

        plsc.subcore_barrier()

        def fire_g(g, b):
            for j in range(K):
                pltpu.async_copy(pe_sh.at[ibs[b].at[j]], rbs[b].at[j], gsem[b])

        def fire_store(g, b):
            pltpu.async_copy(rbs[b], out_hbm.at[pl.ds(base + g * K, K)], ssem[b])

        def fire_idx(g, b):
            pltpu.async_copy(idx_hbm.at[pl.ds(base + g * K, K)], ibs[b], isem[b])

        def wait_store(b):
            pltpu.make_async_copy(rbs[b], out_hbm.at[pl.ds(base, K)], ssem[b]).wait()

        def wait_gathers(b):
            for j in range(K):
                pltpu.make_async_copy(
                    out_hbm.at[base].at[pl.ds(0, BLK)], rbs[b].at[j], gsem[b]
                ).wait()

        def wait_idx(b):
            pltpu.make_async_copy(idx_hbm.at[pl.ds(base, K)], ibs[b], isem[b]).wait()

        pltpu.sync_copy(idx_hbm.at[pl.ds(base, K)], ibs[0])
        fire_g(0, 0)
        fire_idx(1, 1)
        wait_idx(1)
        fire_g(1, 1)
        fire_idx(2, 2)
        wait_gathers(0)
        fire_store(0, 0)
        wait_idx(2)
        fire_g(2, 2)
        fire_idx(3, 0)
        wait_gathers(1)
        fire_store(1, 1)

        def body(t, carry):
            g0 = 2 + NBUF * t
            for db in range(NBUF):
                g = g0 + db
                b = (2 + db) % NBUF
                bn = (b + 1) % NBUF
                bp = (b + 2) % NBUF
                wait_store(bn)
                wait_idx(bn)
                fire_g(g + 1, bn)
                fire_idx(g + 2, bp)
                wait_gathers(b)
                fire_store(g, b)
            return carry

        lax.fori_loop(0, (G - 4) // NBUF, body, 0)

        bL = (G - 2) % NBUF
        wait_store((bL + 1) % NBUF)
        wait_idx((bL + 1) % NBUF)
        fire_g(G - 1, (bL + 1) % NBUF)
        wait_gathers(bL)
        fire_store(G - 2, bL)
        wait_gathers((bL + 1) % NBUF)
        fire_store(G - 1, (bL + 1) % NBUF)
        wait_store((bL + 2) % NBUF)
        wait_store(bL)
        wait_store((bL + 1) % NBUF)

    return k(idx2d, pe)


def kernel(index, pe):
    index = index.astype(jnp.int32)
    b, t = index.shape
    n = b * t
    n_blocks = n // BLK
    idx2d = index.reshape(n_blocks, BLK)
    out = _sc_gather(idx2d, pe, n_blocks)
    return out.reshape(b, t, D_TABLE)
